# baseline (speedup 1.0000x reference)
import functools
import jax
import jax.numpy as jnp
from jax import lax
from jax.experimental import pallas as pl
from jax.experimental.pallas import tpu as pltpu
from jax.experimental.pallas import tpu_sc as plsc

_INTERPRET = False
_USE_SC = True

R = 32
M = R ** 3
F_IN = 4
NC, NS = 2, 16
NW = NC * NS
N_PTS = 100000
N_PAD = 102400
PPW = N_PAD // NW
CHUNK = 128
NCHUNK = PPW // CHUNK
PPW_G = N_PTS // NW
CHUNK_G = 125
NCHUNK_G = PPW_G // CHUNK_G
ROWS_W = M // NS
BLK = 2048
_ALIGN = 16


_PV = 16


def _vox5_body(B):
    def body(t_ref, o_ref):
        base = pl.program_id(0) * B
        r = base + lax.broadcasted_iota(jnp.int32, (B, 1), 0)
        xr = r % R
        win = (t_ref[0, pl.ds(base + _PV - 16, B + 32), :]
               + t_ref[1, pl.ds(base + _PV - 16, B + 32), :])
        vox = win[:, 0:F_IN] / jnp.maximum(win[:, F_IN:F_IN + 1], 1.0)
        pieces = []
        for dx in range(-2, 3):
            mx = ((xr + dx >= 0) & (xr + dx < R)).astype(jnp.float32)
            piece = lax.slice(vox, (16 + dx, 0), (16 + dx + B, F_IN))
            pieces.append(piece * mx)
        o_ref[...] = jnp.concatenate(pieces, axis=1).astype(jnp.bfloat16)

    return body


def _vox5_call(tables):
    B = BLK
    tp = jnp.pad(tables, ((0, 0), (_PV, _PV + _ALIGN), (0, 0)))
    whole = lambda arr: pl.BlockSpec(arr.shape, lambda i: (0,) * arr.ndim)
    return pl.pallas_call(
        _vox5_body(B),
        grid=(M // B,),
        in_specs=[whole(tp)],
        out_specs=pl.BlockSpec((B, 5 * F_IN), lambda i: (i, 0)),
        out_shape=jax.ShapeDtypeStruct((M, 5 * F_IN), jnp.bfloat16),
        interpret=_INTERPRET,
    )(tp)


def _conv_body(k, Rl, B, nin, cins, cout, post=None, xfold=False):
    h = k // 2
    kk = 2 * h + 1
    P = h * (Rl * Rl + Rl + 1)
    BE = B if xfold else B + 2 * h

    def body(*refs):
        xs = refs[:nin]
        ws = refs[nin:2 * nin]
        g_ref, b_ref, o_ref = refs[2 * nin:2 * nin + 3]
        base = pl.program_id(0) * B
        q = (base - (0 if xfold else h)) + lax.broadcasted_iota(
            jnp.int32, (BE, 1), 0)
        zq = q // (Rl * Rl)
        yq = (q // Rl) % Rl
        NY = cout if xfold else kk * 128
        Y = jnp.zeros((BE, NY), jnp.float32)
        for x_ref, w_ref, cin in zip(xs, ws, cins):
            pieces = []
            for dz in range(-h, h + 1):
                for dy in range(-h, h + 1):
                    s = dz * Rl * Rl + dy * Rl
                    mzy = ((zq + dz >= 0) & (zq + dz < Rl)
                           & (yq + dy >= 0) & (yq + dy < Rl)
                           ).astype(jnp.bfloat16)
                    start = P + s - (0 if xfold else h)
                    off = start % _ALIGN
                    astart = start - off
                    win = x_ref[pl.ds(base + astart, BE + _ALIGN), :]
                    piece = lax.slice(win, (off, 0), (off + BE, cin))
                    pieces.append(piece * mzy)
            Xc = jnp.concatenate(pieces, axis=1)
            Y = Y + jnp.dot(Xc, w_ref[...],
                            preferred_element_type=jnp.float32)
        if xfold:
            acc = Y
        else:
            r = base + lax.broadcasted_iota(jnp.int32, (B, 1), 0)
            xr = r % Rl
            acc = jnp.zeros((B, cout), jnp.float32)
            for dxi, dx in enumerate(range(-h, h + 1)):
                mx = ((xr + dx >= 0) & (xr + dx < Rl)).astype(jnp.float32)
                Ys = lax.slice(Y, (h + dx, dxi * 128),
                               (h + dx + B, dxi * 128 + cout))
                acc = acc + Ys * mx
        out = jnp.maximum(acc * g_ref[...] + b_ref[...], 0.0)
        if post is not None:
            out = post(out)
        o_ref[...] = out.astype(o_ref.dtype)

    return body


def _pack_w(w):
    k = w.shape[0]
    cin, cout = w.shape[3], w.shape[4]
    wt = jnp.transpose(w, (0, 1, 3, 2, 4))
    wt = wt.reshape(k * k * cin, k, cout)
    wp = jnp.zeros((k * k * cin, k, 128), jnp.float32)
    wp = wp.at[:, :, :cout].set(wt)
    return wp.reshape(k * k * cin, k * 128).astype(jnp.bfloat16)


def _conv_call(k, Rl, xs, ws, bn, post=None, out_dtype=jnp.bfloat16,
               xfold=False):
    Ml = Rl ** 3
    h = k // 2
    P = h * (Rl * Rl + Rl + 1)
    B = min(BLK, Ml)
    nin = len(xs)
    if xfold:
        cins = [xs[0].shape[-1]]
        cout = ws[0].shape[-1]
        ws_r = [ws[0].astype(jnp.bfloat16)]
    else:
        cins = [w.shape[3] for w in ws]
        cout = ws[0].shape[-1]
        ws_r = [_pack_w(w) for w in ws]
    xps = [jnp.pad(x, ((P, P + _ALIGN), (0, 0))) for x in xs]
    g = bn["gamma"].reshape(1, cout)
    b = bn["beta"].reshape(1, cout)
    body = _conv_body(k, Rl, B, nin, cins, cout, post=post, xfold=xfold)
    whole = lambda arr: pl.BlockSpec(arr.shape, lambda i: (0,) * arr.ndim)
    return pl.pallas_call(
        body,
        grid=(Ml // B,),
        in_specs=[whole(a) for a in (*xps, *ws_r, g, b)],
        out_specs=pl.BlockSpec((B, cout), lambda i: (i, 0)),
        out_shape=jax.ShapeDtypeStruct((Ml, cout), out_dtype),
        compiler_params=pltpu.CompilerParams(
            vmem_limit_bytes=100 * 1024 * 1024),
        interpret=_INTERPRET,
    )(*xps, *ws_r, g, b)


def _pairs(p):
    return [(0, -1), (2, 0)] if p == 0 else [(1, 0)]


def _convt_body(Rh, cout, P):
    Mh = Rh ** 3

    def body(x_ref, w_ref, g_ref, b_ref, o_ref):
        r = lax.broadcasted_iota(jnp.int32, (Mh, 1), 0)
        zc = r // (Rh * Rh)
        yc = (r // Rh) % Rh
        xc = r % Rh
        for pz in (0, 1):
            for py in (0, 1):
                for px in (0, 1):
                    acc = jnp.zeros((Mh, cout), jnp.float32)
                    for tz, dz in _pairs(pz):
                        for ty, dy in _pairs(py):
                            for tx, dx in _pairs(px):
                                s = dz * Rh * Rh + dy * Rh + dx
                                m = ((zc + dz >= 0) & (yc + dy >= 0)
                                     & (xc + dx >= 0)).astype(jnp.bfloat16)
                                Asl = x_ref[pl.ds(P + s, Mh), :]
                                ti = (tz * 3 + ty) * 3 + tx
                                acc = acc + jnp.dot(
                                    Asl * m, w_ref[ti],
                                    preferred_element_type=jnp.float32)
                    out = jnp.maximum(acc * g_ref[...] + b_ref[...], 0.0)
                    o_ref[(pz * 2 + py) * 2 + px] = out.astype(o_ref.dtype)

    return body


def _convt_call(Rl, x, w, bn):
    Rh = Rl // 2
    Mh = Rh ** 3
    P = Rh * Rh + Rh + 1
    cout = w.shape[-1]
    xp = jnp.pad(x, ((P, P), (0, 0)))
    w_r = w.reshape(27, w.shape[3], cout).astype(jnp.bfloat16)
    g = bn["gamma"].reshape(1, cout)
    b = bn["beta"].reshape(1, cout)
    cls = pl.pallas_call(
        _convt_body(Rh, cout, P),
        out_shape=jax.ShapeDtypeStruct((8, Mh, cout), jnp.bfloat16),
        compiler_params=pltpu.CompilerParams(
            vmem_limit_bytes=100 * 1024 * 1024),
        interpret=_INTERPRET,
    )(xp, w_r, g, b)
    full = cls.reshape(2, 2, 2, Rh, Rh, Rh, cout).transpose(3, 0, 4, 1, 5, 2, 6)
    return full.reshape(Rl ** 3, cout)


def _cls_shift(d):
    return (0, 0) if d == -1 else ((1, 0) if d == 0 else (0, 1))


def _s2_body(Rh, cout, P):
    Mh = Rh ** 3

    def body(x_ref, w_ref, g_ref, b_ref, o_ref):
        r = lax.broadcasted_iota(jnp.int32, (Mh, 1), 0)
        zc = r // (Rh * Rh)
        yc = (r // Rh) % Rh
        xc = r % Rh
        acc = jnp.zeros((Mh, cout), jnp.float32)
        for dz in (-1, 0, 1):
            for dy in (-1, 0, 1):
                for dx in (-1, 0, 1):
                    (bz, ez), (by, ey), (bx, ex) = (
                        _cls_shift(dz), _cls_shift(dy), _cls_shift(dx))
                    cls = (bz * 2 + by) * 2 + bx
                    s = ez * Rh * Rh + ey * Rh + ex
                    m = ((zc + ez < Rh) & (yc + ey < Rh)
                         & (xc + ex < Rh)).astype(jnp.bfloat16)
                    Asl = x_ref[cls, pl.ds(P + s, Mh), :]
                    ti = ((dz + 1) * 3 + dy + 1) * 3 + dx + 1
                    acc = acc + jnp.dot(Asl * m, w_ref[ti],
                                        preferred_element_type=jnp.float32)
        out = jnp.maximum(acc * g_ref[...] + b_ref[...], 0.0)
        o_ref[...] = out.astype(o_ref.dtype)

    return body


def _parity_split(x, Rl):
    c = x.shape[-1]
    Rh = Rl // 2
    g = x.reshape(Rh, 2, Rh, 2, Rh, 2, c).transpose(1, 3, 5, 0, 2, 4, 6)
    return g.reshape(8, Rh ** 3, c)


def _conv_s2_call(Rl, x, w, bn):
    Rh = Rl // 2
    Mh = Rh ** 3
    P = Rh * Rh + Rh + 1
    cout = w.shape[-1]
    xp = jnp.pad(_parity_split(x, Rl), ((0, 0), (P, P), (0, 0)))
    w_r = w.reshape(27, w.shape[3], cout).astype(jnp.bfloat16)
    g = bn["gamma"].reshape(1, cout)
    b = bn["beta"].reshape(1, cout)
    return pl.pallas_call(
        _s2_body(Rh, cout, P),
        out_shape=jax.ShapeDtypeStruct((Mh, cout), jnp.bfloat16),
        compiler_params=pltpu.CompilerParams(
            vmem_limit_bytes=100 * 1024 * 1024),
        interpret=_INTERPRET,
    )(xp, w_r, g, b)


def _norm_post(o):
    n = jnp.sqrt(jnp.sum(o * o, axis=1, keepdims=True))
    return o / jnp.maximum(n, 1e-12)


def _sc_scatter(vals, idxs, zer):
    mesh = plsc.VectorSubcoreMesh(core_axis_name="c", subcore_axis_name="s")

    @functools.partial(
        pl.kernel, mesh=mesh,
        out_type=jax.ShapeDtypeStruct((NC, M, 16), jnp.float32),
        compiler_params=pltpu.CompilerParams(use_tc_tiling_on_sc=False),
        scratch_types=[
            pltpu.VMEM((PPW, 16), jnp.float32),
            pltpu.VMEM((NCHUNK, CHUNK), jnp.int32),
            pltpu.VMEM_SHARED((M, 16), jnp.float32),
        ],
    )
    def k(vals_hbm, idx_hbm, zer_hbm, out_hbm, vals_v, idx_v, shared):
        cid = lax.axis_index("c")
        sid = lax.axis_index("s")
        wid = sid * NC + cid
        pltpu.sync_copy(vals_hbm.at[wid], vals_v)
        pltpu.sync_copy(idx_hbm.at[wid], idx_v)
        pltpu.sync_copy(zer_hbm, shared.at[pl.ds(sid * ROWS_W, ROWS_W)])
        plsc.subcore_barrier()
        for j in range(NCHUNK):
            pltpu.sync_copy(vals_v.at[pl.ds(j * CHUNK, CHUNK)],
                            shared.at[idx_v.at[j]], add=True)
        plsc.subcore_barrier()
        pltpu.sync_copy(shared.at[pl.ds(sid * ROWS_W, ROWS_W)],
                        out_hbm.at[cid, pl.ds(sid * ROWS_W, ROWS_W)])

    return k(vals, idxs, zer)


def _sc_gather(table, idxs):
    C = table.shape[1]
    mesh = plsc.VectorSubcoreMesh(core_axis_name="c", subcore_axis_name="s")

    @functools.partial(
        pl.kernel, mesh=mesh,
        out_type=jax.ShapeDtypeStruct((N_PTS, C), jnp.float32),
        compiler_params=pltpu.CompilerParams(use_tc_tiling_on_sc=False),
        scratch_types=[
            pltpu.VMEM((NCHUNK_G, CHUNK_G), jnp.int32),
            pltpu.VMEM((2, CHUNK_G, C), jnp.float32),
            pltpu.SemaphoreType.DMA((2,)),
        ],
    )
    def k(tab_hbm, idx_hbm, out_hbm, idx_v, buf, sem):
        cid = lax.axis_index("c")
        sid = lax.axis_index("s")
        wid = sid * NC + cid
        pltpu.sync_copy(idx_hbm.at[wid], idx_v)
        copies = [pltpu.async_copy(tab_hbm.at[idx_v.at[0]], buf.at[0],
                                   sem.at[0])]
        for j in range(NCHUNK_G):
            if j + 1 < NCHUNK_G:
                copies.append(pltpu.async_copy(
                    tab_hbm.at[idx_v.at[j + 1]], buf.at[(j + 1) % 2],
                    sem.at[(j + 1) % 2]))
            copies[j].wait()
            pltpu.sync_copy(
                buf.at[j % 2],
                out_hbm.at[pl.ds(wid * PPW_G + j * CHUNK_G, CHUNK_G)])

    return k(table, idxs)


def kernel(lidar_feats, lidar_coords, image, py, px, params):
    del image, py, px
    c = lidar_coords.astype(jnp.int32)
    idx = (c[:, 0] * R + c[:, 1]) * R + c[:, 2]
    n = idx.shape[0]

    if _USE_SC:
        idx_p = jnp.concatenate([idx, jnp.zeros((N_PAD - n,), jnp.int32)])
        idxs_s = idx_p.reshape(NW, NCHUNK, CHUNK)
        f16 = jnp.concatenate(
            [lidar_feats, jnp.ones((n, 1), jnp.float32),
             jnp.zeros((n, 11), jnp.float32)], axis=1)
        f16 = jnp.concatenate([f16, jnp.zeros((N_PAD - n, 16), jnp.float32)])
        tables = _sc_scatter(f16.reshape(NW, PPW, 16), idxs_s,
                             jnp.zeros((ROWS_W, 16), jnp.float32))
    else:
        cnt = jax.ops.segment_sum(jnp.ones((n,), jnp.float32), idx,
                                  num_segments=M)
        fsum = jax.ops.segment_sum(lidar_feats, idx, num_segments=M)
        t0 = jnp.concatenate([fsum, cnt[:, None],
                              jnp.zeros((M, 11), jnp.float32)], axis=1)
        tables = jnp.stack([t0, jnp.zeros_like(t0)])

    p = params
    s = p["stem"]
    vox5 = _vox5_call(tables)
    w0p = s["w0"].reshape(125 * F_IN, s["w0"].shape[-1])
    v1 = _conv_call(5, R, [vox5], [w0p], s["bn0"], xfold=True)
    v1 = _conv_call(3, R, [v1], [s["w1"]], s["bn1"])

    def down(hin, Rl, pd):
        hb = _conv_s2_call(Rl, hin, pd["wb"], pd["bnb"])
        return _conv_call(3, Rl // 2, [hb], [pd["w"]], pd["bn"])

    v2 = down(v1, R, p["down1"])
    v4 = down(v2, R // 2, p["down2"])
    v8 = down(v4, R // 4, p["down3"])

    def up(hin, skip, Rl, pu, post=None, out_dtype=jnp.bfloat16):
        ht = _convt_call(Rl, hin, pu["wt"], pu["bnt"])
        ct = pu["wt"].shape[-1]
        w_h = pu["w"][:, :, :, :ct, :]
        w_s = pu["w"][:, :, :, ct:, :]
        return _conv_call(3, Rl, [ht, skip], [w_h, w_s], pu["bn"],
                          post=post, out_dtype=out_dtype)

    v4t = up(v8, v4, R // 4, p["up1"])
    v2t = up(v4t, v2, R // 2, p["up2"])
    table = up(v2t, v1, R, p["up3"], post=_norm_post, out_dtype=jnp.float32)

    if _USE_SC:
        out = _sc_gather(table, idx[:N_PTS].reshape(NW, NCHUNK_G, CHUNK_G))
    else:
        out = table[idx]
    return out


# speedup vs baseline: 1.0491x; 1.0491x over previous
import functools
import jax
import jax.numpy as jnp
from jax import lax
from jax.experimental import pallas as pl
from jax.experimental.pallas import tpu as pltpu
from jax.experimental.pallas import tpu_sc as plsc

_INTERPRET = False
_USE_SC = True

R = 32
M = R ** 3
F_IN = 4
NC, NS = 2, 16
NW = NC * NS
N_PTS = 100000
N_PAD = 102400
PPW = N_PAD // NW
CHUNK = 128
NCHUNK = PPW // CHUNK
PPW_G = N_PTS // NW
CHUNK_G = 125
NCHUNK_G = PPW_G // CHUNK_G
ROWS_W = M // NS
BLK = 2048
_ALIGN = 16


def _vox_body(t_ref, o_ref):
    a = t_ref[0] + t_ref[1]
    o_ref[...] = (a[:, 0:F_IN] / jnp.maximum(a[:, F_IN:F_IN + 1], 1.0)
                  ).astype(jnp.bfloat16)


def _vox_call(tables):
    return pl.pallas_call(
        _vox_body,
        out_shape=jax.ShapeDtypeStruct((M, F_IN), jnp.bfloat16),
        interpret=_INTERPRET,
    )(tables)


_EXT = 8


def _conv_body(k, Rl, B, nin, cins, cout, post=None, pad128=False):
    h = k // 2
    kk = 2 * h + 1
    Ml = Rl ** 3
    P8 = -(-(h * (Rl * Rl + Rl + 1) + _EXT) // _EXT) * _EXT
    BE = B + 2 * _EXT

    def body(*refs):
        xs = refs[:nin]
        ws = refs[nin:2 * nin]
        g_ref, b_ref, o_ref = refs[2 * nin:2 * nin + 3]
        base = 0 if Ml == B else pl.program_id(0) * B
        q = (base - _EXT) + lax.broadcasted_iota(jnp.int32, (BE, 1), 0)
        zq = q // (Rl * Rl)
        yq = (q // Rl) % Rl
        Y = jnp.zeros((BE, kk * 128), jnp.float32)
        for x_ref, w_ref, cin in zip(xs, ws, cins):
            pieces = []
            for dz in range(-h, h + 1):
                for dy in range(-h, h + 1):
                    s = dz * Rl * Rl + dy * Rl
                    mzy = ((zq + dz >= 0) & (zq + dz < Rl)
                           & (yq + dy >= 0) & (yq + dy < Rl)
                           ).astype(jnp.bfloat16)
                    win = x_ref[pl.ds(base + P8 + s - _EXT, BE), :]
                    piece = win * mzy
                    if pad128 and cin < 128:
                        piece = jnp.pad(piece, ((0, 0), (0, 128 - cin)))
                    pieces.append(piece)
            Xc = jnp.concatenate(pieces, axis=1)
            Y = Y + jnp.dot(Xc, w_ref[...],
                            preferred_element_type=jnp.float32)
        r = base + lax.broadcasted_iota(jnp.int32, (B, 1), 0)
        xr = r % Rl
        acc = jnp.zeros((B, cout), jnp.float32)
        for dxi, dx in enumerate(range(-h, h + 1)):
            mx = ((xr + dx >= 0) & (xr + dx < Rl)).astype(jnp.float32)
            Ys = lax.slice(Y, (_EXT + dx, dxi * 128),
                           (_EXT + dx + B, dxi * 128 + cout))
            acc = acc + Ys * mx
        out = jnp.maximum(acc * g_ref[...] + b_ref[...], 0.0)
        if post is not None:
            out = post(out)
        o_ref[...] = out.astype(o_ref.dtype)

    return body


def _pack_w(w, pad128):
    k = w.shape[0]
    cin, cout = w.shape[3], w.shape[4]
    cr = 128 if (pad128 and cin < 128) else cin
    wt = jnp.transpose(w, (0, 1, 3, 2, 4))
    wp = jnp.zeros((k * k, cr, k, 128), jnp.float32)
    wp = wp.at[:, :cin, :, :cout].set(wt.reshape(k * k, cin, k, cout))
    return wp.reshape(k * k * cr, k * 128).astype(jnp.bfloat16)


def _conv_call(k, Rl, xs, ws, bn, post=None, out_dtype=jnp.bfloat16):
    Ml = Rl ** 3
    h = k // 2
    P8 = -(-(h * (Rl * Rl + Rl + 1) + _EXT) // _EXT) * _EXT
    B = min(BLK, Ml)
    nin = len(xs)
    cins = [w.shape[3] for w in ws]
    cout = ws[0].shape[-1]
    pad128 = min(cins) >= 32
    ws_r = [_pack_w(w, pad128) for w in ws]
    xps = [jnp.pad(x, ((P8, P8 + _EXT), (0, 0))) for x in xs]
    g = bn["gamma"].reshape(1, cout)
    b = bn["beta"].reshape(1, cout)
    body = _conv_body(k, Rl, B, nin, cins, cout, post=post, pad128=pad128)
    whole = lambda arr: pl.BlockSpec(arr.shape, lambda i: (0,) * arr.ndim)
    return pl.pallas_call(
        body,
        grid=(Ml // B,),
        in_specs=[whole(a) for a in (*xps, *ws_r, g, b)],
        out_specs=pl.BlockSpec((B, cout), lambda i: (i, 0)),
        out_shape=jax.ShapeDtypeStruct((Ml, cout), out_dtype),
        compiler_params=pltpu.CompilerParams(
            vmem_limit_bytes=100 * 1024 * 1024),
        interpret=_INTERPRET,
    )(*xps, *ws_r, g, b)


def _pairs(p):
    return [(0, -1), (2, 0)] if p == 0 else [(1, 0)]


def _convt_body(Rh, cout, P):
    Mh = Rh ** 3

    def body(x_ref, w_ref, g_ref, b_ref, o_ref):
        r = lax.broadcasted_iota(jnp.int32, (Mh, 1), 0)
        zc = r // (Rh * Rh)
        yc = (r // Rh) % Rh
        xc = r % Rh
        for pz in (0, 1):
            for py in (0, 1):
                for px in (0, 1):
                    acc = jnp.zeros((Mh, cout), jnp.float32)
                    for tz, dz in _pairs(pz):
                        for ty, dy in _pairs(py):
                            for tx, dx in _pairs(px):
                                s = dz * Rh * Rh + dy * Rh + dx
                                m = ((zc + dz >= 0) & (yc + dy >= 0)
                                     & (xc + dx >= 0)).astype(jnp.bfloat16)
                                Asl = x_ref[pl.ds(P + s, Mh), :]
                                ti = (tz * 3 + ty) * 3 + tx
                                acc = acc + jnp.dot(
                                    Asl * m, w_ref[ti],
                                    preferred_element_type=jnp.float32)
                    out = jnp.maximum(acc * g_ref[...] + b_ref[...], 0.0)
                    o_ref[(pz * 2 + py) * 2 + px] = out.astype(o_ref.dtype)

    return body


def _convt_call(Rl, x, w, bn):
    Rh = Rl // 2
    Mh = Rh ** 3
    P = Rh * Rh + Rh + 1
    cout = w.shape[-1]
    xp = jnp.pad(x, ((P, P), (0, 0)))
    w_r = w.reshape(27, w.shape[3], cout).astype(jnp.bfloat16)
    g = bn["gamma"].reshape(1, cout)
    b = bn["beta"].reshape(1, cout)
    cls = pl.pallas_call(
        _convt_body(Rh, cout, P),
        out_shape=jax.ShapeDtypeStruct((8, Mh, cout), jnp.bfloat16),
        compiler_params=pltpu.CompilerParams(
            vmem_limit_bytes=100 * 1024 * 1024),
        interpret=_INTERPRET,
    )(xp, w_r, g, b)
    full = cls.reshape(2, 2, 2, Rh, Rh, Rh, cout).transpose(3, 0, 4, 1, 5, 2, 6)
    return full.reshape(Rl ** 3, cout)


def _cls_shift(d):
    return (0, 0) if d == -1 else ((1, 0) if d == 0 else (0, 1))


def _s2_body(Rh, cout, P):
    Mh = Rh ** 3

    def body(x_ref, w_ref, g_ref, b_ref, o_ref):
        r = lax.broadcasted_iota(jnp.int32, (Mh, 1), 0)
        zc = r // (Rh * Rh)
        yc = (r // Rh) % Rh
        xc = r % Rh
        acc = jnp.zeros((Mh, cout), jnp.float32)
        for dz in (-1, 0, 1):
            for dy in (-1, 0, 1):
                for dx in (-1, 0, 1):
                    (bz, ez), (by, ey), (bx, ex) = (
                        _cls_shift(dz), _cls_shift(dy), _cls_shift(dx))
                    cls = (bz * 2 + by) * 2 + bx
                    s = ez * Rh * Rh + ey * Rh + ex
                    m = ((zc + ez < Rh) & (yc + ey < Rh)
                         & (xc + ex < Rh)).astype(jnp.bfloat16)
                    Asl = x_ref[cls, pl.ds(P + s, Mh), :]
                    ti = ((dz + 1) * 3 + dy + 1) * 3 + dx + 1
                    acc = acc + jnp.dot(Asl * m, w_ref[ti],
                                        preferred_element_type=jnp.float32)
        out = jnp.maximum(acc * g_ref[...] + b_ref[...], 0.0)
        o_ref[...] = out.astype(o_ref.dtype)

    return body


def _parity_split(x, Rl):
    c = x.shape[-1]
    Rh = Rl // 2
    g = x.reshape(Rh, 2, Rh, 2, Rh, 2, c).transpose(1, 3, 5, 0, 2, 4, 6)
    return g.reshape(8, Rh ** 3, c)


def _conv_s2_call(Rl, x, w, bn):
    Rh = Rl // 2
    Mh = Rh ** 3
    P = Rh * Rh + Rh + 1
    cout = w.shape[-1]
    xp = jnp.pad(_parity_split(x, Rl), ((0, 0), (P, P), (0, 0)))
    w_r = w.reshape(27, w.shape[3], cout).astype(jnp.bfloat16)
    g = bn["gamma"].reshape(1, cout)
    b = bn["beta"].reshape(1, cout)
    return pl.pallas_call(
        _s2_body(Rh, cout, P),
        out_shape=jax.ShapeDtypeStruct((Mh, cout), jnp.bfloat16),
        compiler_params=pltpu.CompilerParams(
            vmem_limit_bytes=100 * 1024 * 1024),
        interpret=_INTERPRET,
    )(xp, w_r, g, b)


def _norm_post(o):
    n = jnp.sqrt(jnp.sum(o * o, axis=1, keepdims=True))
    return o / jnp.maximum(n, 1e-12)


def _sc_scatter(vals, idxs, zer):
    mesh = plsc.VectorSubcoreMesh(core_axis_name="c", subcore_axis_name="s")

    @functools.partial(
        pl.kernel, mesh=mesh,
        out_type=jax.ShapeDtypeStruct((NC, M, 16), jnp.float32),
        compiler_params=pltpu.CompilerParams(use_tc_tiling_on_sc=False),
        scratch_types=[
            pltpu.VMEM((PPW, 16), jnp.float32),
            pltpu.VMEM((NCHUNK, CHUNK), jnp.int32),
            pltpu.VMEM_SHARED((M, 16), jnp.float32),
        ],
    )
    def k(vals_hbm, idx_hbm, zer_hbm, out_hbm, vals_v, idx_v, shared):
        cid = lax.axis_index("c")
        sid = lax.axis_index("s")
        wid = sid * NC + cid
        pltpu.sync_copy(vals_hbm.at[wid], vals_v)
        pltpu.sync_copy(idx_hbm.at[wid], idx_v)
        pltpu.sync_copy(zer_hbm, shared.at[pl.ds(sid * ROWS_W, ROWS_W)])
        plsc.subcore_barrier()
        for j in range(NCHUNK):
            pltpu.sync_copy(vals_v.at[pl.ds(j * CHUNK, CHUNK)],
                            shared.at[idx_v.at[j]], add=True)
        plsc.subcore_barrier()
        pltpu.sync_copy(shared.at[pl.ds(sid * ROWS_W, ROWS_W)],
                        out_hbm.at[cid, pl.ds(sid * ROWS_W, ROWS_W)])

    return k(vals, idxs, zer)


def _sc_gather(table, idxs):
    C = table.shape[1]
    mesh = plsc.VectorSubcoreMesh(core_axis_name="c", subcore_axis_name="s")

    @functools.partial(
        pl.kernel, mesh=mesh,
        out_type=jax.ShapeDtypeStruct((N_PTS, C), jnp.float32),
        compiler_params=pltpu.CompilerParams(use_tc_tiling_on_sc=False),
        scratch_types=[
            pltpu.VMEM((NCHUNK_G, CHUNK_G), jnp.int32),
            pltpu.VMEM((2, CHUNK_G, C), jnp.float32),
            pltpu.SemaphoreType.DMA((2,)),
        ],
    )
    def k(tab_hbm, idx_hbm, out_hbm, idx_v, buf, sem):
        cid = lax.axis_index("c")
        sid = lax.axis_index("s")
        wid = sid * NC + cid
        pltpu.sync_copy(idx_hbm.at[wid], idx_v)
        copies = [pltpu.async_copy(tab_hbm.at[idx_v.at[0]], buf.at[0],
                                   sem.at[0])]
        for j in range(NCHUNK_G):
            if j + 1 < NCHUNK_G:
                copies.append(pltpu.async_copy(
                    tab_hbm.at[idx_v.at[j + 1]], buf.at[(j + 1) % 2],
                    sem.at[(j + 1) % 2]))
            copies[j].wait()
            pltpu.sync_copy(
                buf.at[j % 2],
                out_hbm.at[pl.ds(wid * PPW_G + j * CHUNK_G, CHUNK_G)])

    return k(table, idxs)


def kernel(lidar_feats, lidar_coords, image, py, px, params):
    del image, py, px
    c = lidar_coords.astype(jnp.int32)
    idx = (c[:, 0] * R + c[:, 1]) * R + c[:, 2]
    n = idx.shape[0]

    if _USE_SC:
        idx_p = jnp.concatenate([idx, jnp.zeros((N_PAD - n,), jnp.int32)])
        idxs_s = idx_p.reshape(NW, NCHUNK, CHUNK)
        f16 = jnp.concatenate(
            [lidar_feats, jnp.ones((n, 1), jnp.float32),
             jnp.zeros((n, 11), jnp.float32)], axis=1)
        f16 = jnp.concatenate([f16, jnp.zeros((N_PAD - n, 16), jnp.float32)])
        tables = _sc_scatter(f16.reshape(NW, PPW, 16), idxs_s,
                             jnp.zeros((ROWS_W, 16), jnp.float32))
    else:
        cnt = jax.ops.segment_sum(jnp.ones((n,), jnp.float32), idx,
                                  num_segments=M)
        fsum = jax.ops.segment_sum(lidar_feats, idx, num_segments=M)
        t0 = jnp.concatenate([fsum, cnt[:, None],
                              jnp.zeros((M, 11), jnp.float32)], axis=1)
        tables = jnp.stack([t0, jnp.zeros_like(t0)])

    p = params
    s = p["stem"]
    vox = _vox_call(tables)
    v1 = _conv_call(5, R, [vox], [s["w0"]], s["bn0"])
    v1 = _conv_call(3, R, [v1], [s["w1"]], s["bn1"])

    def down(hin, Rl, pd):
        hb = _conv_s2_call(Rl, hin, pd["wb"], pd["bnb"])
        return _conv_call(3, Rl // 2, [hb], [pd["w"]], pd["bn"])

    v2 = down(v1, R, p["down1"])
    v4 = down(v2, R // 2, p["down2"])
    v8 = down(v4, R // 4, p["down3"])

    def up(hin, skip, Rl, pu, post=None, out_dtype=jnp.bfloat16):
        ht = _convt_call(Rl, hin, pu["wt"], pu["bnt"])
        ct = pu["wt"].shape[-1]
        w_h = pu["w"][:, :, :, :ct, :]
        w_s = pu["w"][:, :, :, ct:, :]
        return _conv_call(3, Rl, [ht, skip], [w_h, w_s], pu["bn"],
                          post=post, out_dtype=out_dtype)

    v4t = up(v8, v4, R // 4, p["up1"])
    v2t = up(v4t, v2, R // 2, p["up2"])
    table = up(v2t, v1, R, p["up3"], post=_norm_post, out_dtype=jnp.float32)

    if _USE_SC:
        out = _sc_gather(table, idx[:N_PTS].reshape(NW, NCHUNK_G, CHUNK_G))
    else:
        out = table[idx]
    return out


# speedup vs baseline: 1.1138x; 1.0617x over previous
import functools
import jax
import jax.numpy as jnp
from jax import lax
from jax.experimental import pallas as pl
from jax.experimental.pallas import tpu as pltpu
from jax.experimental.pallas import tpu_sc as plsc

_INTERPRET = False
_USE_SC = True

R = 32
M = R ** 3
F_IN = 4
NC, NS = 2, 16
NW = NC * NS
N_PTS = 100000
N_PAD = 102400
PPW = N_PAD // NW
CHUNK = 128
NCHUNK = PPW // CHUNK
PPW_G = N_PTS // NW
CHUNK_G = 125
NCHUNK_G = PPW_G // CHUNK_G
ROWS_W = M // NS
BLK = 2048
_ALIGN = 16


def _vox_body(t_ref, o_ref):
    a = t_ref[0] + t_ref[1]
    o_ref[...] = (a[:, 0:F_IN] / jnp.maximum(a[:, F_IN:F_IN + 1], 1.0)
                  ).astype(jnp.bfloat16)


def _vox_call(tables):
    return pl.pallas_call(
        _vox_body,
        out_shape=jax.ShapeDtypeStruct((M, F_IN), jnp.bfloat16),
        interpret=_INTERPRET,
    )(tables)


def _conv_body(k, Rl, B, nin, cins, cout, post=None):
    h = k // 2
    kk = 2 * h + 1
    P = h * (Rl * Rl + Rl + 1)
    BE = B + 2 * h

    def body(*refs):
        xs = refs[:nin]
        ws = refs[nin:2 * nin]
        g_ref, b_ref, o_ref = refs[2 * nin:2 * nin + 3]
        base = pl.program_id(0) * B
        q = (base - h) + lax.broadcasted_iota(jnp.int32, (BE, 1), 0)
        zq = q // (Rl * Rl)
        yq = (q // Rl) % Rl
        r = base + lax.broadcasted_iota(jnp.int32, (B, 1), 0)
        xr = r % Rl
        acc = jnp.zeros((B, cout), jnp.float32)
        Y = jnp.zeros((BE, kk * 128), jnp.float32)
        for x_ref, w_ref, cin in zip(xs, ws, cins):
            pieces = []
            for dz in range(-h, h + 1):
                for dy in range(-h, h + 1):
                    s = dz * Rl * Rl + dy * Rl
                    mzy = ((zq + dz >= 0) & (zq + dz < Rl)
                           & (yq + dy >= 0) & (yq + dy < Rl)
                           ).astype(jnp.bfloat16)
                    start = P - h + s
                    off = start % _ALIGN
                    astart = start - off
                    win = x_ref[pl.ds(base + astart, BE + _ALIGN), :]
                    piece = lax.slice(win, (off, 0), (off + BE, cin))
                    pieces.append(piece * mzy)
            Xc = jnp.concatenate(pieces, axis=1)
            Y = Y + jnp.dot(Xc, w_ref[...],
                            preferred_element_type=jnp.float32)
        for dxi, dx in enumerate(range(-h, h + 1)):
            mx = ((xr + dx >= 0) & (xr + dx < Rl)).astype(jnp.float32)
            Ys = lax.slice(Y, (h + dx, dxi * 128),
                           (h + dx + B, dxi * 128 + cout))
            acc = acc + Ys * mx
        out = jnp.maximum(acc * g_ref[...] + b_ref[...], 0.0)
        if post is not None:
            out = post(out)
        o_ref[...] = out.astype(o_ref.dtype)

    return body


def _pack_w(w):
    k = w.shape[0]
    cin, cout = w.shape[3], w.shape[4]
    wt = jnp.transpose(w, (0, 1, 3, 2, 4))
    wt = wt.reshape(k * k * cin, k, cout)
    wp = jnp.zeros((k * k * cin, k, 128), jnp.float32)
    wp = wp.at[:, :, :cout].set(wt)
    return wp.reshape(k * k * cin, k * 128).astype(jnp.bfloat16)


def _conv_call(k, Rl, xs, ws, bn, post=None, out_dtype=jnp.bfloat16):
    Ml = Rl ** 3
    h = k // 2
    P = h * (Rl * Rl + Rl + 1)
    B = min(BLK, Ml)
    nin = len(xs)
    cins = [w.shape[3] for w in ws]
    cout = ws[0].shape[-1]
    ws_r = [_pack_w(w) for w in ws]
    xps = [jnp.pad(x, ((P, P + _ALIGN), (0, 0))) for x in xs]
    g = bn["gamma"].reshape(1, cout)
    b = bn["beta"].reshape(1, cout)
    body = _conv_body(k, Rl, B, nin, cins, cout, post=post)
    whole = lambda arr: pl.BlockSpec(arr.shape, lambda i: (0,) * arr.ndim)
    return pl.pallas_call(
        body,
        grid=(Ml // B,),
        in_specs=[whole(a) for a in (*xps, *ws_r, g, b)],
        out_specs=pl.BlockSpec((B, cout), lambda i: (i, 0)),
        out_shape=jax.ShapeDtypeStruct((Ml, cout), out_dtype),
        compiler_params=pltpu.CompilerParams(
            vmem_limit_bytes=100 * 1024 * 1024),
        interpret=_INTERPRET,
    )(*xps, *ws_r, g, b)


def _pairs(p):
    return [(0, -1), (2, 0)] if p == 0 else [(1, 0)]


def _convt_body(Rh, cout, P):
    Mh = Rh ** 3

    def body(x_ref, w_ref, g_ref, b_ref, o_ref):
        r = lax.broadcasted_iota(jnp.int32, (Mh, 1), 0)
        zc = r // (Rh * Rh)
        yc = (r // Rh) % Rh
        xc = r % Rh
        for pz in (0, 1):
            for py in (0, 1):
                for px in (0, 1):
                    acc = jnp.zeros((Mh, cout), jnp.float32)
                    for tz, dz in _pairs(pz):
                        for ty, dy in _pairs(py):
                            for tx, dx in _pairs(px):
                                s = dz * Rh * Rh + dy * Rh + dx
                                m = ((zc + dz >= 0) & (yc + dy >= 0)
                                     & (xc + dx >= 0)).astype(jnp.bfloat16)
                                Asl = x_ref[pl.ds(P + s, Mh), :]
                                ti = (tz * 3 + ty) * 3 + tx
                                acc = acc + jnp.dot(
                                    Asl * m, w_ref[ti],
                                    preferred_element_type=jnp.float32)
                    out = jnp.maximum(acc * g_ref[...] + b_ref[...], 0.0)
                    o_ref[(pz * 2 + py) * 2 + px] = out.astype(o_ref.dtype)

    return body


def _convt_call(Rl, x, w, bn):
    Rh = Rl // 2
    Mh = Rh ** 3
    P = Rh * Rh + Rh + 1
    cout = w.shape[-1]
    xp = jnp.pad(x, ((P, P), (0, 0)))
    w_r = w.reshape(27, w.shape[3], cout).astype(jnp.bfloat16)
    g = bn["gamma"].reshape(1, cout)
    b = bn["beta"].reshape(1, cout)
    cls = pl.pallas_call(
        _convt_body(Rh, cout, P),
        out_shape=jax.ShapeDtypeStruct((8, Mh, cout), jnp.bfloat16),
        compiler_params=pltpu.CompilerParams(
            vmem_limit_bytes=100 * 1024 * 1024),
        interpret=_INTERPRET,
    )(xp, w_r, g, b)
    full = cls.reshape(2, 2, 2, Rh, Rh, Rh, cout).transpose(3, 0, 4, 1, 5, 2, 6)
    return full.reshape(Rl ** 3, cout)


def _cls_shift(d):
    return (0, 0) if d == -1 else ((1, 0) if d == 0 else (0, 1))


def _s2_body(Rh, cout, P):
    Mh = Rh ** 3

    def body(x_ref, w_ref, g_ref, b_ref, o_ref):
        r = lax.broadcasted_iota(jnp.int32, (Mh, 1), 0)
        zc = r // (Rh * Rh)
        yc = (r // Rh) % Rh
        xc = r % Rh
        acc = jnp.zeros((Mh, cout), jnp.float32)
        for dz in (-1, 0, 1):
            for dy in (-1, 0, 1):
                for dx in (-1, 0, 1):
                    (bz, ez), (by, ey), (bx, ex) = (
                        _cls_shift(dz), _cls_shift(dy), _cls_shift(dx))
                    cls = (bz * 2 + by) * 2 + bx
                    s = ez * Rh * Rh + ey * Rh + ex
                    m = ((zc + ez < Rh) & (yc + ey < Rh)
                         & (xc + ex < Rh)).astype(jnp.bfloat16)
                    Asl = x_ref[cls, pl.ds(P + s, Mh), :]
                    ti = ((dz + 1) * 3 + dy + 1) * 3 + dx + 1
                    acc = acc + jnp.dot(Asl * m, w_ref[ti],
                                        preferred_element_type=jnp.float32)
        out = jnp.maximum(acc * g_ref[...] + b_ref[...], 0.0)
        o_ref[...] = out.astype(o_ref.dtype)

    return body


def _parity_split(x, Rl):
    c = x.shape[-1]
    Rh = Rl // 2
    g = x.reshape(Rh, 2, Rh, 2, Rh, 2, c).transpose(1, 3, 5, 0, 2, 4, 6)
    return g.reshape(8, Rh ** 3, c)


def _conv_s2_call(Rl, x, w, bn):
    Rh = Rl // 2
    Mh = Rh ** 3
    P = Rh * Rh + Rh + 1
    cout = w.shape[-1]
    xp = jnp.pad(_parity_split(x, Rl), ((0, 0), (P, P), (0, 0)))
    w_r = w.reshape(27, w.shape[3], cout).astype(jnp.bfloat16)
    g = bn["gamma"].reshape(1, cout)
    b = bn["beta"].reshape(1, cout)
    return pl.pallas_call(
        _s2_body(Rh, cout, P),
        out_shape=jax.ShapeDtypeStruct((Mh, cout), jnp.bfloat16),
        compiler_params=pltpu.CompilerParams(
            vmem_limit_bytes=100 * 1024 * 1024),
        interpret=_INTERPRET,
    )(xp, w_r, g, b)


def _norm_post(o):
    n = jnp.sqrt(jnp.sum(o * o, axis=1, keepdims=True))
    return o / jnp.maximum(n, 1e-12)


def _sc_scatter(vals, idxs, zer):
    mesh = plsc.VectorSubcoreMesh(core_axis_name="c", subcore_axis_name="s")

    @functools.partial(
        pl.kernel, mesh=mesh,
        out_type=jax.ShapeDtypeStruct((NC, M, 16), jnp.float32),
        compiler_params=pltpu.CompilerParams(use_tc_tiling_on_sc=False),
        scratch_types=[
            pltpu.VMEM((PPW, 16), jnp.float32),
            pltpu.VMEM((NCHUNK, CHUNK), jnp.int32),
            pltpu.VMEM_SHARED((M, 16), jnp.float32),
        ],
    )
    def k(vals_hbm, idx_hbm, zer_hbm, out_hbm, vals_v, idx_v, shared):
        cid = lax.axis_index("c")
        sid = lax.axis_index("s")
        wid = sid * NC + cid
        pltpu.sync_copy(vals_hbm.at[wid], vals_v)
        pltpu.sync_copy(idx_hbm.at[wid], idx_v)
        pltpu.sync_copy(zer_hbm, shared.at[pl.ds(sid * ROWS_W, ROWS_W)])
        plsc.subcore_barrier()
        for j in range(NCHUNK):
            pltpu.sync_copy(vals_v.at[pl.ds(j * CHUNK, CHUNK)],
                            shared.at[idx_v.at[j]], add=True)
        plsc.subcore_barrier()
        pltpu.sync_copy(shared.at[pl.ds(sid * ROWS_W, ROWS_W)],
                        out_hbm.at[cid, pl.ds(sid * ROWS_W, ROWS_W)])

    return k(vals, idxs, zer)


def _sc_gather(table, idxs):
    C = table.shape[1]
    mesh = plsc.VectorSubcoreMesh(core_axis_name="c", subcore_axis_name="s")

    @functools.partial(
        pl.kernel, mesh=mesh,
        out_type=jax.ShapeDtypeStruct((N_PTS, C), jnp.float32),
        compiler_params=pltpu.CompilerParams(use_tc_tiling_on_sc=False),
        scratch_types=[
            pltpu.VMEM((NCHUNK_G, CHUNK_G), jnp.int32),
            pltpu.VMEM((2, CHUNK_G, C), jnp.float32),
            pltpu.SemaphoreType.DMA((2,)),
        ],
    )
    def k(tab_hbm, idx_hbm, out_hbm, idx_v, buf, sem):
        cid = lax.axis_index("c")
        sid = lax.axis_index("s")
        wid = sid * NC + cid
        pltpu.sync_copy(idx_hbm.at[wid], idx_v)
        copies = [pltpu.async_copy(tab_hbm.at[idx_v.at[0]], buf.at[0],
                                   sem.at[0])]
        for j in range(NCHUNK_G):
            if j + 1 < NCHUNK_G:
                copies.append(pltpu.async_copy(
                    tab_hbm.at[idx_v.at[j + 1]], buf.at[(j + 1) % 2],
                    sem.at[(j + 1) % 2]))
            copies[j].wait()
            pltpu.sync_copy(
                buf.at[j % 2],
                out_hbm.at[pl.ds(wid * PPW_G + j * CHUNK_G, CHUNK_G)])

    return k(table, idxs)


def kernel(lidar_feats, lidar_coords, image, py, px, params):
    del image, py, px
    c = lidar_coords.astype(jnp.int32)
    idx = (c[:, 0] * R + c[:, 1]) * R + c[:, 2]
    n = idx.shape[0]

    if _USE_SC:
        idx_p = jnp.concatenate([idx, jnp.zeros((N_PAD - n,), jnp.int32)])
        idxs_s = idx_p.reshape(NW, NCHUNK, CHUNK)
        f16 = jnp.concatenate(
            [lidar_feats, jnp.ones((n, 1), jnp.float32),
             jnp.zeros((n, 11), jnp.float32)], axis=1)
        f16 = jnp.concatenate([f16, jnp.zeros((N_PAD - n, 16), jnp.float32)])
        tables = _sc_scatter(f16.reshape(NW, PPW, 16), idxs_s,
                             jnp.zeros((ROWS_W, 16), jnp.float32))
    else:
        cnt = jax.ops.segment_sum(jnp.ones((n,), jnp.float32), idx,
                                  num_segments=M)
        fsum = jax.ops.segment_sum(lidar_feats, idx, num_segments=M)
        t0 = jnp.concatenate([fsum, cnt[:, None],
                              jnp.zeros((M, 11), jnp.float32)], axis=1)
        tables = jnp.stack([t0, jnp.zeros_like(t0)])

    p = params
    s = p["stem"]
    vox = _vox_call(tables)
    v1 = _conv_call(5, R, [vox], [s["w0"]], s["bn0"])
    v1 = _conv_call(3, R, [v1], [s["w1"]], s["bn1"])

    def down(hin, Rl, pd):
        hb = _conv_s2_call(Rl, hin, pd["wb"], pd["bnb"])
        return _conv_call(3, Rl // 2, [hb], [pd["w"]], pd["bn"])

    v2 = down(v1, R, p["down1"])
    v4 = down(v2, R // 2, p["down2"])
    v8 = down(v4, R // 4, p["down3"])

    def up(hin, skip, Rl, pu, post=None, out_dtype=jnp.bfloat16):
        ht = _convt_call(Rl, hin, pu["wt"], pu["bnt"])
        ct = pu["wt"].shape[-1]
        w_h = pu["w"][:, :, :, :ct, :]
        w_s = pu["w"][:, :, :, ct:, :]
        return _conv_call(3, Rl, [ht, skip], [w_h, w_s], pu["bn"],
                          post=post, out_dtype=out_dtype)

    v4t = up(v8, v4, R // 4, p["up1"])
    v2t = up(v4t, v2, R // 2, p["up2"])
    table = up(v2t, v1, R, p["up3"], post=_norm_post, out_dtype=jnp.float32)

    if _USE_SC:
        out = _sc_gather(table, idx[:N_PTS].reshape(NW, NCHUNK_G, CHUNK_G))
    else:
        out = table[idx]
    return out
